# Initial kernel scaffold; baseline (speedup 1.0000x reference)
import jax
import jax.numpy as jnp
from jax.experimental import pallas as pl


def kernel(encoder_out, target, target_mask, W, b):
    raise NotImplementedError("write your pallas kernel here")


# speedup vs baseline: 1.1869x; 1.1869x over previous
import functools

import jax
import jax.numpy as jnp
from jax.experimental import pallas as pl
from jax.experimental.pallas import tpu as pltpu

B, S, D, V = 4, 2048, 2048, 1000
M = B * S
TILE_M = 512
NUM_TILES = M // TILE_M


def _head_kernel(x_ref, w_ref, b_ref, tgt_ref, logits_ref, probs_ref, loss_ref,
                 acc_ref):
    i = pl.program_id(0)

    x = x_ref[...]
    w = w_ref[...]
    logits = jax.lax.dot_general(
        x, w, (((1,), (1,)), ((), ())),
        preferred_element_type=jnp.float32)
    logits = logits + b_ref[...]
    logits_ref[...] = logits

    m = jnp.max(logits, axis=-1, keepdims=True)
    ex = jnp.exp(logits - m)
    s = jnp.sum(ex, axis=-1, keepdims=True)
    probs_ref[...] = ex * (1.0 / s)

    t = tgt_ref[0, pl.ds(i * TILE_M, TILE_M)]
    t2 = t[:, None]
    onehot = (jax.lax.broadcasted_iota(jnp.int32, (TILE_M, V), 1) == t2)
    tgt_logit = jnp.sum(jnp.where(onehot, logits, 0.0), axis=-1, keepdims=True)
    lse = m + jnp.log(s)
    valid = t2 >= 0
    nll = jnp.where(valid, lse - tgt_logit, 0.0)

    tile_sum = jnp.sum(nll)
    tile_cnt = jnp.sum(valid.astype(jnp.float32))

    @pl.when(i == 0)
    def _init():
        acc_ref[0] = 0.0
        acc_ref[1] = 0.0

    acc_ref[0] += tile_sum
    acc_ref[1] += tile_cnt

    @pl.when(i == NUM_TILES - 1)
    def _fin():
        val = acc_ref[0] / jnp.maximum(acc_ref[1], 1.0)
        loss_ref[...] = jnp.broadcast_to(val, (1, 1))


@jax.jit
def _head(x, w, b, tgt):
    logits, probs, loss = pl.pallas_call(
        _head_kernel,
        grid=(NUM_TILES,),
        in_specs=[
            pl.BlockSpec((TILE_M, D), lambda i: (i, 0)),
            pl.BlockSpec((V, D), lambda i: (0, 0)),
            pl.BlockSpec((1, V), lambda i: (0, 0)),
            pl.BlockSpec((1, M), lambda i: (0, 0)),
        ],
        out_specs=[
            pl.BlockSpec((TILE_M, V), lambda i: (i, 0)),
            pl.BlockSpec((TILE_M, V), lambda i: (i, 0)),
            pl.BlockSpec((1, 1), lambda i: (0, 0)),
        ],
        out_shape=[
            jax.ShapeDtypeStruct((M, V), jnp.float32),
            jax.ShapeDtypeStruct((M, V), jnp.float32),
            jax.ShapeDtypeStruct((1, 1), jnp.float32),
        ],
        scratch_shapes=[pltpu.SMEM((2,), jnp.float32)],
    )(x, w, b, tgt)
    return logits, probs, loss


def kernel(encoder_out, target, target_mask, W, b):
    x = encoder_out.reshape(M, D)
    tgt = jnp.where(target_mask, target, -1).astype(jnp.int32).reshape(1, M)
    logits, probs, loss = _head(x, W, b.reshape(1, V), tgt)
    return (logits.reshape(B, S, V), probs.reshape(B, S, V), loss[0, 0])
